# baseline (speedup 1.0000x reference)
import functools
import jax
import jax.numpy as jnp
from jax import lax
from jax.experimental import pallas as pl
from jax.experimental.pallas import tpu as pltpu
from jax.experimental.pallas import tpu_sc as plsc

N_SEG = 10000
NC = 2
NS = 16
NW = NC * NS
L = 16

SB = 256
SB_SHIFT = 8
XB = 128
XB_SHIFT = 7
CB = 64
CB_SHIFT = 6
DRAIN = 64
DRAIN_SHIFT = 6
BS_ITERS = 19


def _make_sc_call(E, D):
    CH = E // NW
    NBLK = (CH + XB - 1) // XB
    NFULL = CH // XB
    TAIL = CH - NFULL * XB
    NGRP = CH // L
    NCH = D // L
    mesh = plsc.VectorSubcoreMesh(core_axis_name="c", subcore_axis_name="s")

    @functools.partial(
        pl.kernel,
        out_type=jax.ShapeDtypeStruct((N_SEG * D,), jnp.float32),
        mesh=mesh,
        scratch_types=[
            pltpu.VMEM((CH + 2 * L,), jnp.int32),
            pltpu.VMEM((CH + 2 * L,), jnp.int32),
            pltpu.VMEM((NBLK * L,), jnp.int32),
            pltpu.VMEM((L,), jnp.int32),
            pltpu.VMEM((XB * D,), jnp.float32),
            pltpu.VMEM((XB * D,), jnp.float32),
            pltpu.VMEM((XB * D,), jnp.float32),
            pltpu.VMEM((CB + 2 * L,), jnp.int32),
            pltpu.VMEM((CB * D,), jnp.float32),
            pltpu.VMEM((SB * D,), jnp.float32),
            pltpu.SemaphoreType.DMA,
            pltpu.SemaphoreType.DMA,
            pltpu.SemaphoreType.DMA,
            pltpu.SemaphoreType.DMA,
            pltpu.SemaphoreType.DMA,
        ],
        compiler_params=pltpu.CompilerParams(use_tc_tiling_on_sc=False,
                                            needs_layout_passes=False),
    )
    def sc_call(x_hbm, idx_hbm, out_hbm, idx_v, b_v, bc_v, head_v,
                x0, x1, x2, probe_v, cx_v, staging,
                sem0, sem1, sem2, sem_b, sem_f):
        bufs = (x0, x1, x2)
        sems = (sem0, sem1, sem2)

        def copy_wait(src, dst):
            pltpu.async_copy(src, dst, sem_f).wait()

        def sread(ref, i):
            return ref[pl.ds(i, L)][0]

        cid = lax.axis_index("c")
        sid = lax.axis_index("s")
        w = sid * NC + cid
        base = w * CH

        pltpu.async_copy(idx_hbm.at[pl.ds(base, CH)], idx_v.at[pl.ds(L, CH)],
                         sem_b)
        offp = pl.multiple_of(jnp.where(w == 0, 0, base - 8), 8)
        pltpu.sync_copy(idx_hbm.at[pl.ds(offp, 8)], head_v.at[pl.ds(0, 8)])
        offn = pl.multiple_of(jnp.where(w == NW - 1, E - 8, base + CH), 8)
        pltpu.sync_copy(idx_hbm.at[pl.ds(offn, 8)], head_v.at[pl.ds(8, 8)])

        def zrow(r, _):
            staging[pl.ds(r * L, L)] = jnp.zeros((L,), jnp.float32)
            return 0
        lax.fori_loop(0, SB * NCH, zrow, 0)
        pltpu.make_async_copy(idx_hbm.at[pl.ds(base, CH)],
                              idx_v.at[pl.ds(L, CH)], sem_b).wait()

        hv = head_v[pl.ds(0, L)]
        fw = sread(idx_v, L)
        prev = hv[7]
        o_w = jnp.where(w == 0, 0, jnp.where(prev == fw, fw + 1, fw))
        fnext = hv[8]
        lastc = sread(idx_v, L + CH - 1)
        o_next = jnp.where(w == NW - 1, N_SEG,
                           jnp.where(lastc == fnext, fnext + 1, fnext))

        idx_v[pl.ds(0, L)] = jnp.full((L,), fw, jnp.int32)

        for j in range(3):
            pltpu.async_copy(
                x_hbm.at[pl.ds((base + j * XB) * D, XB * D)], bufs[j],
                sems[j])

        lanes = jnp.arange(L, dtype=jnp.int32)

        def scan_grp(g, nb):
            v = idx_v[pl.ds(L + g * L, L)]
            vp = idx_v[pl.ds(L - 1 + g * L, L)]
            m = v != vp
            cnt = plsc.all_reduce_population_count(m)[0]

            def app_body(_, mnb):
                m_, nb_ = mnb
                pos = plsc.all_reduce_ffs(m_)[0]
                b_v[pl.ds(nb_, L)] = jnp.full((L,), g * L, jnp.int32) + pos
                return jnp.logical_and(m_, lanes != pos), nb_ + 1

            _, nb = lax.fori_loop(0, cnt, app_body, (m, nb))
            return nb

        GPB = XB // L

        def scan_blk(blk, nb):
            nb0 = nb
            for gg in range(GPB):
                nb = scan_grp(blk * GPB + gg, nb)
            bc_v[pl.ds(blk * L, L)] = jnp.full((L,), nb - nb0, jnp.int32)
            return nb

        nb = lax.fori_loop(0, NFULL, scan_blk, jnp.int32(0))
        nb0 = nb
        for gg in range(TAIL // L):
            nb = scan_grp(NFULL * GPB + gg, nb)
        bc_v[pl.ds(NFULL * L, L)] = jnp.full((L,), nb - nb0, jnp.int32)
        b_v[pl.ds(nb, L)] = jnp.full((L,), CH, jnp.int32)

        neg_init = jnp.full((L,), -jnp.inf, jnp.float32)

        def flush_body(_, fb_):
            copy_wait(staging,
                      out_hbm.at[pl.ds(pl.multiple_of((o_w + fb_) * D, 8),
                                       SB * D)])
            lax.fori_loop(0, SB * NCH, zrow, 0)
            return fb_ + SB

        def emit(cur, accs, fb):
            nfl = jnp.maximum(0, (cur - o_w - fb) >> SB_SHIFT)
            fb = lax.fori_loop(0, nfl, flush_body, fb)

            @pl.when(cur >= o_w)
            def _():
                rel = cur - o_w - fb
                for c in range(NCH):
                    staging[pl.ds(rel * D + c * L, L)] = accs[c]

            return fb

        def accum(buf, lo, hi, accs):
            def rbody(r, a):
                return tuple(
                    jnp.maximum(a[c], buf[pl.ds(r * D + c * L, L)])
                    for c in range(NCH))
            return plsc.parallel_loop(lo, hi, 1, unroll=4, carry=accs)(rbody)

        def do_block(b, s, nrows, shift, buf, carry):
            accs = carry[:NCH]
            cur = carry[NCH]
            fb = carry[NCH + 1]
            jp = carry[NCH + 2]
            rp = carry[NCH + 3]
            n_ev = sread(bc_v, b * L)

            def ev_body(_, ec):
                accs = ec[:NCH]
                cur = ec[NCH]
                fb = ec[NCH + 1]
                jp = ec[NCH + 2]
                rp = ec[NCH + 3]
                p = sread(b_v, jp)
                accs = accum(buf, rp - s + shift, p - s + shift, accs)
                fb = emit(cur, accs, fb)
                cur = sread(idx_v, L + p)
                return (neg_init,) * NCH + (cur, fb, jp + 1, p)

            ec = lax.fori_loop(0, n_ev, ev_body,
                               accs + (cur, fb, jp, rp))
            accs = ec[:NCH]
            cur = ec[NCH]
            fb = ec[NCH + 1]
            jp = ec[NCH + 2]
            rp = ec[NCH + 3]
            accs = accum(buf, rp - s + shift, shift + nrows, accs)
            return accs + (cur, fb, jp, jnp.int32(s + nrows))

        carry = (neg_init,) * NCH + (fw, jnp.int32(0), jnp.int32(0),
                                     jnp.int32(0))

        def outer(k, carry):
            b0 = 3 * k
            for j in range(3):
                pltpu.make_async_copy(x_hbm.at[pl.ds(0, XB * D)], bufs[j],
                                      sems[j]).wait()
                carry = do_block(b0 + j, (b0 + j) * XB, XB, 0, bufs[j], carry)
                offp_ = jnp.minimum(base + (b0 + j + 3) * XB,
                                    base + CH - XB)
                pltpu.async_copy(x_hbm.at[pl.ds(offp_ * D, XB * D)], bufs[j],
                                 sems[j])
            return carry

        NRING = (NFULL // 3) * 3
        carry = lax.fori_loop(0, NFULL // 3, outer, carry)
        for b in range(NRING, NBLK):
            jj = b % 3
            sb_ = b * XB
            nrows_ = min(XB, CH - sb_)
            shift_ = max(0, sb_ - (CH - XB))
            pltpu.make_async_copy(x_hbm.at[pl.ds(0, XB * D)], bufs[jj],
                                  sems[jj]).wait()
            carry = do_block(b, sb_, nrows_, shift_, bufs[jj], carry)
        for b in range(NBLK, NRING + 3):
            jj = b % 3
            pltpu.make_async_copy(x_hbm.at[pl.ds(0, XB * D)], bufs[jj],
                                  sems[jj]).wait()

        accs = carry[:NCH]
        cur = carry[NCH]
        fb = carry[NCH + 1]

        offc = pl.multiple_of(jnp.where(w == NW - 1, E - CB, base + CH), 8)
        copy_wait(idx_hbm.at[pl.ds(offc, CB)], probe_v.at[pl.ds(0, CB)])

        def led_body(i, la):
            led_, alleq_ = la
            eq = sread(probe_v, i) == cur
            cont = jnp.logical_and(alleq_, eq)
            return led_ + jnp.where(cont, 1, 0), cont

        led, alleq = lax.fori_loop(0, CB, led_body,
                                   (jnp.int32(0), jnp.bool_(True)))

        lo0 = base + CH + CB
        nbs = jnp.where(jnp.logical_and(alleq, w != NW - 1), BS_ITERS, 0)

        def bs_body(_, lohi):
            lo, hi = lohi
            mid = (lo + hi) >> 1
            midc = jnp.minimum(mid, E - 8)
            moff = pl.multiple_of(midc & ~jnp.int32(7), 8)
            copy_wait(idx_hbm.at[pl.ds(moff, 8)], probe_v.at[pl.ds(CB, 8)])
            val = sread(probe_v, CB + (midc & jnp.int32(7)))
            gt = jnp.logical_or(val != cur, lo >= hi)
            hi = jnp.where(gt, jnp.minimum(mid, hi), hi)
            lo = jnp.where(gt, lo, mid + 1)
            return lo, hi

        lo_f, _ = lax.fori_loop(0, nbs, bs_body, (lo0, jnp.int32(E)))
        n_cont = jnp.where(w == NW - 1, 0,
                           jnp.where(alleq, lo_f - (base + CH), led))
        n_cblk = (n_cont + CB - 1) >> CB_SHIFT

        def cblk_body(k, accs):
            r0 = base + CH + k * CB
            r0c = pl.multiple_of(jnp.minimum(r0, E - CB), 8)
            copy_wait(x_hbm.at[pl.ds(pl.multiple_of(r0c * D, 8), CB * D)],
                      cx_v)
            shift = r0 - r0c
            cnt = jnp.minimum(CB, n_cont - k * CB)
            return accum(cx_v, shift, shift + cnt, accs)

        accs = lax.fori_loop(0, n_cblk, cblk_body, accs)

        fb = emit(cur, accs, fb)
        rel_end = o_next - o_w
        fb = lax.fori_loop(0, (rel_end - fb) >> SB_SHIFT, flush_body, fb)
        rem = rel_end - fb

        def dfull_body(k, _):
            off = k * DRAIN
            copy_wait(
                staging.at[pl.ds(off * D, DRAIN * D)],
                out_hbm.at[pl.ds(pl.multiple_of((o_w + fb + off) * D, 8),
                                 DRAIN * D)])
            return 0

        lax.fori_loop(0, rem >> DRAIN_SHIFT, dfull_body, 0)
        toff = (rem >> DRAIN_SHIFT) * DRAIN

        def dtail_start(k, _):
            pltpu.async_copy(
                staging.at[pl.ds((toff + k) * D, D)],
                out_hbm.at[pl.ds(pl.multiple_of((o_w + fb + toff + k) * D, 8),
                                 D)], sem_f)
            return 0

        def dtail_wait(k, _):
            pltpu.make_async_copy(
                staging.at[pl.ds(0, D)],
                out_hbm.at[pl.ds(0, D)], sem_f).wait()
            return 0

        lax.fori_loop(0, rem - toff, dtail_start, 0)
        lax.fori_loop(0, rem - toff, dtail_wait, 0)

    return sc_call


def kernel(x, index):
    E, D = x.shape
    sc_call = _make_sc_call(E, D)
    return sc_call(x.reshape(E * D), index).reshape(N_SEG, D)


# speedup vs baseline: 1.0322x; 1.0322x over previous
import functools
import jax
import jax.numpy as jnp
from jax import lax
from jax.experimental import pallas as pl
from jax.experimental.pallas import tpu as pltpu
from jax.experimental.pallas import tpu_sc as plsc

N_SEG = 10000
NC = 2
NS = 16
NW = NC * NS
L = 16

SB = 128
SB_SHIFT = 7
XB = 128
XB_SHIFT = 7
CB = 64
CB_SHIFT = 6
DRAIN = 64
DRAIN_SHIFT = 6
BS_ITERS = 19


def _make_sc_call(E, D):
    CH = E // NW
    NBLK = (CH + XB - 1) // XB
    NFULL = CH // XB
    TAIL = CH - NFULL * XB
    NGRP = CH // L
    NCH = D // L
    mesh = plsc.VectorSubcoreMesh(core_axis_name="c", subcore_axis_name="s")

    @functools.partial(
        pl.kernel,
        out_type=jax.ShapeDtypeStruct((N_SEG * D,), jnp.float32),
        mesh=mesh,
        scratch_types=[
            pltpu.VMEM((CH + 2 * L,), jnp.int32),
            pltpu.VMEM((CH + 2 * L,), jnp.int32),
            pltpu.VMEM((NBLK * L,), jnp.int32),
            pltpu.VMEM((L,), jnp.int32),
            pltpu.VMEM((XB * D,), jnp.float32),
            pltpu.VMEM((XB * D,), jnp.float32),
            pltpu.VMEM((XB * D,), jnp.float32),
            pltpu.VMEM((XB * D,), jnp.float32),
            pltpu.VMEM((CB + 2 * L,), jnp.int32),
            pltpu.VMEM((CB * D,), jnp.float32),
            pltpu.VMEM((SB * D,), jnp.float32),
            pltpu.SemaphoreType.DMA,
            pltpu.SemaphoreType.DMA,
            pltpu.SemaphoreType.DMA,
            pltpu.SemaphoreType.DMA,
            pltpu.SemaphoreType.DMA,
            pltpu.SemaphoreType.DMA,
        ],
        compiler_params=pltpu.CompilerParams(use_tc_tiling_on_sc=False,
                                            needs_layout_passes=False),
    )
    def sc_call(x_hbm, idx_hbm, out_hbm, idx_v, b_v, bc_v, head_v,
                x0, x1, x2, x3, probe_v, cx_v, staging,
                sem0, sem1, sem2, sem3, sem_b, sem_f):
        bufs = (x0, x1, x2, x3)
        sems = (sem0, sem1, sem2, sem3)

        def copy_wait(src, dst):
            pltpu.async_copy(src, dst, sem_f).wait()

        def sread(ref, i):
            return ref[pl.ds(i, L)][0]

        cid = lax.axis_index("c")
        sid = lax.axis_index("s")
        w = sid * NC + cid
        base = w * CH

        pltpu.async_copy(idx_hbm.at[pl.ds(base, CH)], idx_v.at[pl.ds(L, CH)],
                         sem_b)
        offp = pl.multiple_of(jnp.where(w == 0, 0, base - 8), 8)
        pltpu.sync_copy(idx_hbm.at[pl.ds(offp, 8)], head_v.at[pl.ds(0, 8)])
        offn = pl.multiple_of(jnp.where(w == NW - 1, E - 8, base + CH), 8)
        pltpu.sync_copy(idx_hbm.at[pl.ds(offn, 8)], head_v.at[pl.ds(8, 8)])

        def zrow(r, _):
            staging[pl.ds(r * L, L)] = jnp.zeros((L,), jnp.float32)
            return 0
        lax.fori_loop(0, SB * NCH, zrow, 0)
        pltpu.make_async_copy(idx_hbm.at[pl.ds(base, CH)],
                              idx_v.at[pl.ds(L, CH)], sem_b).wait()

        hv = head_v[pl.ds(0, L)]
        fw = sread(idx_v, L)
        prev = hv[7]
        o_w = jnp.where(w == 0, 0, jnp.where(prev == fw, fw + 1, fw))
        fnext = hv[8]
        lastc = sread(idx_v, L + CH - 1)
        o_next = jnp.where(w == NW - 1, N_SEG,
                           jnp.where(lastc == fnext, fnext + 1, fnext))

        idx_v[pl.ds(0, L)] = jnp.full((L,), fw, jnp.int32)

        for j in range(4):
            pltpu.async_copy(
                x_hbm.at[pl.ds((base + j * XB) * D, XB * D)], bufs[j],
                sems[j])

        lanes = jnp.arange(L, dtype=jnp.int32)

        def scan_grp(g, nb):
            v = idx_v[pl.ds(L + g * L, L)]
            vp = idx_v[pl.ds(L - 1 + g * L, L)]
            m = v != vp
            cnt = plsc.all_reduce_population_count(m)[0]

            def app_body(_, mnb):
                m_, nb_ = mnb
                pos = plsc.all_reduce_ffs(m_)[0]
                b_v[pl.ds(nb_, L)] = jnp.full((L,), g * L, jnp.int32) + pos
                return jnp.logical_and(m_, lanes != pos), nb_ + 1

            _, nb = lax.fori_loop(0, cnt, app_body, (m, nb))
            return nb

        GPB = XB // L

        def scan_blk(blk, nb):
            nb0 = nb
            for gg in range(GPB):
                nb = scan_grp(blk * GPB + gg, nb)
            bc_v[pl.ds(blk * L, L)] = jnp.full((L,), nb - nb0, jnp.int32)
            return nb

        nb = lax.fori_loop(0, NFULL, scan_blk, jnp.int32(0))
        nb0 = nb
        for gg in range(TAIL // L):
            nb = scan_grp(NFULL * GPB + gg, nb)
        bc_v[pl.ds(NFULL * L, L)] = jnp.full((L,), nb - nb0, jnp.int32)
        b_v[pl.ds(nb, L)] = jnp.full((L,), CH, jnp.int32)

        neg_init = jnp.full((L,), -jnp.inf, jnp.float32)

        def flush_body(_, fb_):
            copy_wait(staging,
                      out_hbm.at[pl.ds(pl.multiple_of((o_w + fb_) * D, 8),
                                       SB * D)])
            lax.fori_loop(0, SB * NCH, zrow, 0)
            return fb_ + SB

        def emit(cur, accs, fb):
            nfl = jnp.maximum(0, (cur - o_w - fb) >> SB_SHIFT)
            fb = lax.fori_loop(0, nfl, flush_body, fb)

            @pl.when(cur >= o_w)
            def _():
                rel = cur - o_w - fb
                for c in range(NCH):
                    staging[pl.ds(rel * D + c * L, L)] = accs[c]

            return fb

        def accum(buf, lo, hi, accs):
            def rbody(r, a):
                return tuple(
                    jnp.maximum(a[c], buf[pl.ds(r * D + c * L, L)])
                    for c in range(NCH))
            return plsc.parallel_loop(lo, hi, 1, unroll=4, carry=accs)(rbody)

        def do_block(b, s, nrows, shift, buf, carry):
            accs = carry[:NCH]
            cur = carry[NCH]
            fb = carry[NCH + 1]
            jp = carry[NCH + 2]
            rp = carry[NCH + 3]
            n_ev = sread(bc_v, b * L)

            def ev_body(_, ec):
                accs = ec[:NCH]
                cur = ec[NCH]
                fb = ec[NCH + 1]
                jp = ec[NCH + 2]
                rp = ec[NCH + 3]
                p = sread(b_v, jp)
                accs = accum(buf, rp - s + shift, p - s + shift, accs)
                fb = emit(cur, accs, fb)
                cur = sread(idx_v, L + p)
                return (neg_init,) * NCH + (cur, fb, jp + 1, p)

            ec = lax.fori_loop(0, n_ev, ev_body,
                               accs + (cur, fb, jp, rp))
            accs = ec[:NCH]
            cur = ec[NCH]
            fb = ec[NCH + 1]
            jp = ec[NCH + 2]
            rp = ec[NCH + 3]
            accs = accum(buf, rp - s + shift, shift + nrows, accs)
            return accs + (cur, fb, jp, jnp.int32(s + nrows))

        carry = (neg_init,) * NCH + (fw, jnp.int32(0), jnp.int32(0),
                                     jnp.int32(0))

        def outer(k, carry):
            b0 = 4 * k
            for j in range(4):
                pltpu.make_async_copy(x_hbm.at[pl.ds(0, XB * D)], bufs[j],
                                      sems[j]).wait()
                carry = do_block(b0 + j, (b0 + j) * XB, XB, 0, bufs[j], carry)
                offp_ = jnp.minimum(base + (b0 + j + 4) * XB,
                                    base + CH - XB)
                pltpu.async_copy(x_hbm.at[pl.ds(offp_ * D, XB * D)], bufs[j],
                                 sems[j])
            return carry

        NRING = (NFULL // 4) * 4
        carry = lax.fori_loop(0, NFULL // 4, outer, carry)
        for b in range(NRING, NBLK):
            jj = b % 4
            sb_ = b * XB
            nrows_ = min(XB, CH - sb_)
            shift_ = max(0, sb_ - (CH - XB))
            pltpu.make_async_copy(x_hbm.at[pl.ds(0, XB * D)], bufs[jj],
                                  sems[jj]).wait()
            carry = do_block(b, sb_, nrows_, shift_, bufs[jj], carry)
        for b in range(NBLK, NRING + 4):
            jj = b % 4
            pltpu.make_async_copy(x_hbm.at[pl.ds(0, XB * D)], bufs[jj],
                                  sems[jj]).wait()

        accs = carry[:NCH]
        cur = carry[NCH]
        fb = carry[NCH + 1]

        offc = pl.multiple_of(jnp.where(w == NW - 1, E - CB, base + CH), 8)
        copy_wait(idx_hbm.at[pl.ds(offc, CB)], probe_v.at[pl.ds(0, CB)])

        def led_body(i, la):
            led_, alleq_ = la
            eq = sread(probe_v, i) == cur
            cont = jnp.logical_and(alleq_, eq)
            return led_ + jnp.where(cont, 1, 0), cont

        led, alleq = lax.fori_loop(0, CB, led_body,
                                   (jnp.int32(0), jnp.bool_(True)))

        lo0 = base + CH + CB
        nbs = jnp.where(jnp.logical_and(alleq, w != NW - 1), BS_ITERS, 0)

        def bs_body(_, lohi):
            lo, hi = lohi
            mid = (lo + hi) >> 1
            midc = jnp.minimum(mid, E - 8)
            moff = pl.multiple_of(midc & ~jnp.int32(7), 8)
            copy_wait(idx_hbm.at[pl.ds(moff, 8)], probe_v.at[pl.ds(CB, 8)])
            val = sread(probe_v, CB + (midc & jnp.int32(7)))
            gt = jnp.logical_or(val != cur, lo >= hi)
            hi = jnp.where(gt, jnp.minimum(mid, hi), hi)
            lo = jnp.where(gt, lo, mid + 1)
            return lo, hi

        lo_f, _ = lax.fori_loop(0, nbs, bs_body, (lo0, jnp.int32(E)))
        n_cont = jnp.where(w == NW - 1, 0,
                           jnp.where(alleq, lo_f - (base + CH), led))
        n_cblk = (n_cont + CB - 1) >> CB_SHIFT

        def cblk_body(k, accs):
            r0 = base + CH + k * CB
            r0c = pl.multiple_of(jnp.minimum(r0, E - CB), 8)
            copy_wait(x_hbm.at[pl.ds(pl.multiple_of(r0c * D, 8), CB * D)],
                      cx_v)
            shift = r0 - r0c
            cnt = jnp.minimum(CB, n_cont - k * CB)
            return accum(cx_v, shift, shift + cnt, accs)

        accs = lax.fori_loop(0, n_cblk, cblk_body, accs)

        fb = emit(cur, accs, fb)
        rel_end = o_next - o_w
        fb = lax.fori_loop(0, (rel_end - fb) >> SB_SHIFT, flush_body, fb)
        rem = rel_end - fb

        def dfull_body(k, _):
            off = k * DRAIN
            copy_wait(
                staging.at[pl.ds(off * D, DRAIN * D)],
                out_hbm.at[pl.ds(pl.multiple_of((o_w + fb + off) * D, 8),
                                 DRAIN * D)])
            return 0

        lax.fori_loop(0, rem >> DRAIN_SHIFT, dfull_body, 0)
        toff = (rem >> DRAIN_SHIFT) * DRAIN

        def dtail_start(k, _):
            pltpu.async_copy(
                staging.at[pl.ds((toff + k) * D, D)],
                out_hbm.at[pl.ds(pl.multiple_of((o_w + fb + toff + k) * D, 8),
                                 D)], sem_f)
            return 0

        def dtail_wait(k, _):
            pltpu.make_async_copy(
                staging.at[pl.ds(0, D)],
                out_hbm.at[pl.ds(0, D)], sem_f).wait()
            return 0

        lax.fori_loop(0, rem - toff, dtail_start, 0)
        lax.fori_loop(0, rem - toff, dtail_wait, 0)

    return sc_call


def kernel(x, index):
    E, D = x.shape
    sc_call = _make_sc_call(E, D)
    return sc_call(x.reshape(E * D), index).reshape(N_SEG, D)
